# baseline (speedup 1.0000x reference)
import functools

import jax
import jax.numpy as jnp
from jax import lax
from jax.experimental import pallas as pl
from jax.experimental.pallas import tpu as pltpu
from jax.experimental.pallas import tpu_sc as plsc

_NC = 2
_NS = 16
_NW = _NC * _NS


def _mesh():
    return plsc.VectorSubcoreMesh(
        core_axis_name="c", subcore_axis_name="s", num_cores=_NC, num_subcores=_NS
    )


@functools.cache
def _transpose_fn(V, D, CHV):
    n_chunks = V // CHV
    n_iter = -(-n_chunks // _NW)

    @functools.partial(
        pl.kernel,
        out_type=jax.ShapeDtypeStruct((V, D), jnp.float32),
        mesh=_mesh(),
        scratch_types=[
            pltpu.VMEM((2, D, CHV), jnp.float32),
            pltpu.VMEM((2, CHV, D), jnp.float32),
            pltpu.SemaphoreType.DMA((2,)),
            pltpu.SemaphoreType.DMA((2,)),
        ],
        compiler_params=pltpu.CompilerParams(use_tc_tiling_on_sc=False, needs_layout_passes=False),
    )
    def k1(tt_hbm, t2_hbm, in_v, out_v, sin, sout):
        w = lax.axis_index("s") * _NC + lax.axis_index("c")
        iota = lax.iota(jnp.int32, 16)

        def chunk_of(t):
            return t * _NW + w

        def start_in(t, b):
            c = chunk_of(t)

            @pl.when(c < n_chunks)
            def _():
                pltpu.async_copy(
                    tt_hbm.at[:, pl.ds(c * CHV, CHV)], in_v.at[b], sin.at[b]
                )

        def wait_in(b):
            pltpu.make_async_copy(
                tt_hbm.at[:, pl.ds(0, CHV)], in_v.at[b], sin.at[b]
            ).wait()

        def start_out(t, b):
            c = chunk_of(t)
            pltpu.async_copy(
                out_v.at[b], t2_hbm.at[pl.ds(c * CHV, CHV)], sout.at[b]
            )

        def wait_out(b):
            pltpu.make_async_copy(
                out_v.at[b], t2_hbm.at[pl.ds(0, CHV)], sout.at[b]
            ).wait()

        def transpose(b):
            def jbody(j, carry):
                col = j * 16 + iota
                for d in range(D):
                    vals = plsc.load_gather(in_v.at[b], [jnp.full((16,), d, jnp.int32), col])
                    plsc.store_scatter(out_v.at[b], [col, jnp.full((16,), d, jnp.int32)], vals)
                return carry

            lax.fori_loop(0, CHV // 16, jbody, 0)

        start_in(0, 0)
        start_in(1, 1)
        for t0, b0 in ((0, 0), (1, 1)):
            wait_in(b0)
            transpose(b0)
            start_out(t0, b0)
            start_in(t0 + 2, b0)

        def body(tt, carry):
            for b in range(2):
                t = tt * 2 + b

                @pl.when(chunk_of(t) < n_chunks)
                def _():
                    wait_in(b)
                    wait_out(b)
                    transpose(b)
                    start_out(t, b)
                    start_in(t + 2, b)
            return carry

        lax.fori_loop(1, (n_iter + 1) // 2 + 1, body, 0)

        wait_out(0)
        wait_out(1)

    return k1


@functools.cache
def _gather_fn(V, D, SEQ, BATCH):
    QB = 1024
    n_units = SEQ * (BATCH // QB)
    upw = n_units // _NW
    nbh = QB // 128

    @functools.partial(
        pl.kernel,
        out_type=jax.ShapeDtypeStruct((SEQ, D // 8, BATCH // 128, 8, 128), jnp.float32),
        mesh=_mesh(),
        scratch_types=[
            pltpu.VMEM((2, QB), jnp.int32),
            pltpu.VMEM((2, QB, D), jnp.float32),
            pltpu.VMEM((nbh, D, 128), jnp.float32),
            pltpu.SemaphoreType.DMA((2,)),
            pltpu.SemaphoreType.DMA((2,)),
            pltpu.SemaphoreType.DMA,
        ],
        compiler_params=pltpu.CompilerParams(use_tc_tiling_on_sc=False, needs_layout_passes=False),
    )
    def k2(idx_hbm, t2_hbm, x5_hbm, idx_v, rows_v, tr_v, sidx, srow, sout):
        w = lax.axis_index("s") * _NC + lax.axis_index("c")
        iota = lax.iota(jnp.int32, 16)

        def start_unit(t, b):
            u = w * upw + t
            off = u * QB
            pltpu.async_copy(idx_hbm.at[pl.ds(off, QB)], idx_v.at[b], sidx.at[b])

        def fire_gather(b):
            pltpu.make_async_copy(
                idx_hbm.at[pl.ds(0, QB)], idx_v.at[b], sidx.at[b]
            ).wait()
            pltpu.async_copy(t2_hbm.at[idx_v.at[b]], rows_v.at[b], srow.at[b])

        def wait_gather(b):
            pltpu.make_async_copy(
                t2_hbm.at[idx_v.at[b]], rows_v.at[b], srow.at[b]
            ).wait()

        def process(t, b):
            u = w * upw + t
            l = u // (BATCH // QB)
            q = u % (BATCH // QB)
            for bh in range(nbh):
                def jbody(j, carry):
                    row = bh * 128 + j * 16 + iota
                    for d in range(D):
                        dcol = jnp.full((16,), d, jnp.int32)
                        vals = plsc.load_gather(rows_v.at[b], [row, dcol])
                        plsc.store_scatter(
                            tr_v.at[bh], [dcol, j * 16 + iota], vals
                        )
                    return carry

                lax.fori_loop(0, 128 // 16, jbody, 0)
            for dh in range(D // 8):
                pltpu.async_copy(
                    tr_v.at[:, pl.ds(dh * 8, 8)],
                    x5_hbm.at[l, dh, pl.ds(q * nbh, nbh)],
                    sout,
                )

        def drain_out():
            for dh in range(D // 8):
                pltpu.make_async_copy(
                    tr_v.at[:, pl.ds(dh * 8, 8)],
                    x5_hbm.at[0, dh, pl.ds(0, nbh)],
                    sout,
                ).wait()

        start_unit(0, 0)
        fire_gather(0)
        start_unit(1, 1)
        fire_gather(1)

        def body(t, carry):
            for b in range(2):
                tb = t * 2 + b

                @pl.when(tb < upw)
                def _():
                    wait_gather(b)

                    @pl.when(tb + 2 < upw)
                    def _():
                        start_unit(tb + 2, b)

                    @pl.when(tb >= 1)
                    def _():
                        drain_out()
                    process(tb, b)

                    @pl.when(tb + 2 < upw)
                    def _():
                        fire_gather(b)
            return carry

        lax.fori_loop(0, (upw + 1) // 2, body, 0)
        drain_out()

    return k2


def kernel(indices, table):
    bsz, seq = indices.shape
    V, D = table.shape
    idxT = indices.T.reshape(bsz * seq).astype(jnp.int32)
    tableT = table.T
    t2 = _transpose_fn(V, D, 400)(tableT)
    x5 = _gather_fn(V, D, seq, bsz)(idxT, t2)
    return x5.transpose(2, 4, 0, 1, 3).reshape(bsz, seq, D)
